# Initial kernel scaffold; baseline (speedup 1.0000x reference)
import functools

import jax
import jax.numpy as jnp
from jax import lax
from jax.experimental import pallas as pl
from jax.experimental.pallas import tpu as pltpu
from jax.experimental.pallas import tpu_sc as plsc

D = 32
NC = 2
NS = 16
NW = NC * NS
CHUNK = 1024


def _sc_gather_fn(n, per_w, n_chunks):
    mesh = plsc.VectorSubcoreMesh(core_axis_name="c", subcore_axis_name="s")

    @functools.partial(
        pl.kernel,
        mesh=mesh,
        out_type=jax.ShapeDtypeStruct((n, D), jnp.float32),
        scratch_types=[
            pltpu.VMEM((CHUNK,), jnp.int32),
            pltpu.VMEM((CHUNK, D), jnp.float32),
            pltpu.SemaphoreType.DMA,
        ],
    )
    def sc_gather(idx_hbm, table_hbm, out_hbm, idx_v, rows_v, sem):
        wid = lax.axis_index("s") * NC + lax.axis_index("c")
        base = wid * per_w

        def body(i, carry):
            off = base + i * CHUNK
            pltpu.sync_copy(idx_hbm.at[pl.ds(off, CHUNK)], idx_v)
            pltpu.async_copy(table_hbm.at[idx_v], rows_v, sem).wait()
            pltpu.sync_copy(rows_v, out_hbm.at[pl.ds(off, CHUNK)])
            return carry

        lax.fori_loop(0, n_chunks, body, 0)

    return sc_gather


def kernel(sentence, table):
    B, L = sentence.shape
    n = B * L
    assert n % (NW * CHUNK) == 0
    per_w = n // NW
    n_chunks = per_w // CHUNK
    idx = sentence.reshape(n).astype(jnp.int32)
    out = _sc_gather_fn(n, per_w, n_chunks)(idx, table)
    return out.reshape(B, L, D)


# speedup vs baseline: 4.8102x; 4.8102x over previous
import functools

import jax
import jax.numpy as jnp
from jax import lax
from jax.experimental import pallas as pl
from jax.experimental.pallas import tpu as pltpu
from jax.experimental.pallas import tpu_sc as plsc

D = 32
NC = 2
NS = 16
NW = NC * NS
CHUNK = 1024


def _sc_gather_fn(n, per_w, n_chunks):
    mesh = plsc.VectorSubcoreMesh(core_axis_name="c", subcore_axis_name="s")

    @functools.partial(
        pl.kernel,
        mesh=mesh,
        out_type=jax.ShapeDtypeStruct((n, D), jnp.float32),
        scratch_types=[
            pltpu.VMEM((CHUNK,), jnp.int32),
            pltpu.VMEM((CHUNK, D), jnp.float32),
            pltpu.SemaphoreType.DMA,
        ],
        compiler_params=pltpu.CompilerParams(use_tc_tiling_on_sc=False),
    )
    def sc_gather(idx_hbm, table_hbm, out_hbm, idx_v, rows_v, sem):
        wid = lax.axis_index("s") * NC + lax.axis_index("c")
        base = wid * per_w

        def body(i, carry):
            off = base + i * CHUNK
            pltpu.sync_copy(idx_hbm.at[pl.ds(off, CHUNK)], idx_v)
            pltpu.async_copy(table_hbm.at[idx_v], rows_v, sem).wait()
            pltpu.sync_copy(rows_v, out_hbm.at[pl.ds(off, CHUNK)])
            return carry

        lax.fori_loop(0, n_chunks, body, 0)

    return sc_gather


def kernel(sentence, table):
    B, L = sentence.shape
    n = B * L
    assert n % (NW * CHUNK) == 0
    per_w = n // NW
    n_chunks = per_w // CHUNK
    idx = sentence.reshape(n).astype(jnp.int32)
    out = _sc_gather_fn(n, per_w, n_chunks)(idx, table)
    return out.reshape(B, L, D)


# speedup vs baseline: 5.0409x; 1.0480x over previous
import functools

import jax
import jax.numpy as jnp
from jax import lax
from jax.experimental import pallas as pl
from jax.experimental.pallas import tpu as pltpu
from jax.experimental.pallas import tpu_sc as plsc

D = 32
NC = 2
NS = 16
NW = NC * NS
CHUNK = 800
NBUF = 4


def _sc_gather_fn(n, per_w, n_chunks):
    mesh = plsc.VectorSubcoreMesh(core_axis_name="c", subcore_axis_name="s")
    rounds = n_chunks // NBUF

    scratch = (
        [pltpu.VMEM((CHUNK,), jnp.int32) for _ in range(NBUF)]
        + [pltpu.VMEM((CHUNK, D), jnp.float32) for _ in range(NBUF)]
        + [pltpu.SemaphoreType.DMA for _ in range(3 * NBUF)]
    )

    @functools.partial(
        pl.kernel,
        mesh=mesh,
        out_type=jax.ShapeDtypeStruct((n, D), jnp.float32),
        scratch_types=scratch,
        compiler_params=pltpu.CompilerParams(use_tc_tiling_on_sc=False),
    )
    def sc_gather(idx_hbm, table_hbm, out_hbm, *bufs):
        idx_v = bufs[:NBUF]
        rows_v = bufs[NBUF:2 * NBUF]
        isem = bufs[2 * NBUF:3 * NBUF]
        gsem = bufs[3 * NBUF:4 * NBUF]
        wsem = bufs[4 * NBUF:5 * NBUF]

        wid = lax.axis_index("s") * NC + lax.axis_index("c")
        base = wid * per_w

        def idx_copy(b, c):
            off = base + c * CHUNK
            return pltpu.make_async_copy(
                idx_hbm.at[pl.ds(off, CHUNK)], idx_v[b], isem[b])

        def gather_copy(b):
            return pltpu.make_async_copy(table_hbm.at[idx_v[b]], rows_v[b], gsem[b])

        def wb_copy(b, c):
            off = base + c * CHUNK
            return pltpu.make_async_copy(
                rows_v[b], out_hbm.at[pl.ds(off, CHUNK)], wsem[b])

        for b in range(NBUF):
            idx_copy(b, b).start()
        for b in range(NBUF):
            idx_copy(b, b).wait()
            gather_copy(b).start()

        def body(r, carry):
            c0 = r * NBUF
            for b in range(NBUF):
                gather_copy(b).wait()
                wb_copy(b, c0 + b).start()

                @pl.when(r + 1 < rounds)
                def _():
                    idx_copy(b, c0 + NBUF + b).start()

            @pl.when(r + 1 < rounds)
            def _():
                for b in range(NBUF):
                    idx_copy(b, c0 + NBUF + b).wait()
                    wb_copy(b, c0 + b).wait()
                    gather_copy(b).start()

            return carry

        lax.fori_loop(0, rounds, body, 0)

        for b in range(NBUF):
            wb_copy(b, (rounds - 1) * NBUF + b).wait()

    return sc_gather


def kernel(sentence, table):
    B, L = sentence.shape
    n = B * L
    assert n % (NW * CHUNK * NBUF) == 0
    per_w = n // NW
    n_chunks = per_w // CHUNK
    idx = sentence.reshape(n).astype(jnp.int32)
    out = _sc_gather_fn(n, per_w, n_chunks)(idx, table)
    return out.reshape(B, L, D)
